# Initial kernel scaffold; baseline (speedup 1.0000x reference)
import jax
import jax.numpy as jnp
from jax.experimental import pallas as pl


def kernel(triples, entity_re, entity_im, relation_re, relation_im):
    raise NotImplementedError("write your pallas kernel here")


# speedup vs baseline: 2.0993x; 2.0993x over previous
import functools

import jax
import jax.numpy as jnp
from jax import lax
from jax.experimental import pallas as pl
from jax.experimental.pallas import tpu as pltpu
from jax.experimental.pallas import tpu_sc as plsc

NC = 2
NS = 16
L = 16
NW = NC * NS


def kernel(triples, entity_re, entity_im, relation_re, relation_im):
    B = triples.shape[0]
    D = entity_re.shape[1]
    h_idx = triples[:, 0]
    r_idx = triples[:, 1]
    t_idx = triples[:, 2]

    CH = 128
    per_w = B // NW
    n_ch = per_w // CH
    n_sl = D // L

    mesh = plsc.VectorSubcoreMesh(core_axis_name="c", subcore_axis_name="s")

    @functools.partial(
        pl.kernel,
        mesh=mesh,
        compiler_params=pltpu.CompilerParams(needs_layout_passes=False),
        out_type=jax.ShapeDtypeStruct((B,), jnp.float32),
        scratch_types=[
            pltpu.VMEM((CH,), jnp.int32),
            pltpu.VMEM((CH,), jnp.int32),
            pltpu.VMEM((CH,), jnp.int32),
            pltpu.VMEM((CH, D), jnp.float32),
            pltpu.VMEM((CH, D), jnp.float32),
            pltpu.VMEM((CH, D), jnp.float32),
            pltpu.VMEM((CH, D), jnp.float32),
            pltpu.VMEM((CH, D), jnp.float32),
            pltpu.VMEM((CH, D), jnp.float32),
            pltpu.VMEM((CH,), jnp.float32),
            pltpu.SemaphoreType.DMA,
        ],
    )
    def scmk(hidx_hbm, ridx_hbm, tidx_hbm, ere_hbm, eim_hbm, rre_hbm, rim_hbm,
             out_hbm, ih_v, ir_v, it_v, hre_v, him_v, rre_v, rim_v, tre_v,
             tim_v, sc_v, sem):
        wid = lax.axis_index("s") * NC + lax.axis_index("c")
        wbase = wid * per_w
        lanes = lax.iota(jnp.int32, L)

        def chunk_body(c, carry):
            base = wbase + c * CH
            pltpu.sync_copy(hidx_hbm.at[pl.ds(base, CH)], ih_v)
            pltpu.sync_copy(ridx_hbm.at[pl.ds(base, CH)], ir_v)
            pltpu.sync_copy(tidx_hbm.at[pl.ds(base, CH)], it_v)
            cps = [
                pltpu.async_copy(ere_hbm.at[ih_v], hre_v, sem),
                pltpu.async_copy(eim_hbm.at[ih_v], him_v, sem),
                pltpu.async_copy(rre_hbm.at[ir_v], rre_v, sem),
                pltpu.async_copy(rim_hbm.at[ir_v], rim_v, sem),
                pltpu.async_copy(ere_hbm.at[it_v], tre_v, sem),
                pltpu.async_copy(eim_hbm.at[it_v], tim_v, sem),
            ]
            for cp in cps:
                cp.wait()

            def group_body(g, carry2):
                rows = g * L + lanes
                acc = jnp.zeros((L,), jnp.float32)
                for d in range(D):
                    cols = (lanes + d) & (D - 1)
                    idx = [rows, cols]
                    hre = plsc.load_gather(hre_v, idx)
                    him = plsc.load_gather(him_v, idx)
                    rre = plsc.load_gather(rre_v, idx)
                    rim = plsc.load_gather(rim_v, idx)
                    tre = plsc.load_gather(tre_v, idx)
                    tim = plsc.load_gather(tim_v, idx)
                    acc = acc + (hre * rre - him * rim) * tre \
                              + (hre * rim + him * rre) * tim
                sc_v[pl.ds(g * L, L)] = acc
                return carry2

            lax.fori_loop(0, CH // L, group_body, 0)
            pltpu.sync_copy(sc_v, out_hbm.at[pl.ds(base, CH)])
            return carry

        lax.fori_loop(0, n_ch, chunk_body, 0)

    return scmk(h_idx, r_idx, t_idx, entity_re, entity_im, relation_re,
                relation_im)
